# baseline (speedup 1.0000x reference)
import functools

import jax
import jax.numpy as jnp
from jax import lax
from jax.experimental import pallas as pl
from jax.experimental.pallas import tpu as pltpu
from jax.experimental.pallas import tpu_sc as plsc

_N = 2048
_S = 1024
_R = 512
_H = 1024
_OUT = 256
_K = 8
_RW = (_N - _S) // 32


def _sc_body(px_h, py_h, vx_h, vy_h, out_h, pxv, pyv, vxv, vyv, og):
    wid = lax.axis_index("s") * 2 + lax.axis_index("c")
    pltpu.sync_copy(px_h, pxv)
    pltpu.sync_copy(py_h, pyv)
    pltpu.sync_copy(vx_h, vxv)
    pltpu.sync_copy(vy_h, vyv)
    base = _S + wid * _RW
    lanei = lax.iota(jnp.int32, 16)
    inf = jnp.float32(jnp.inf)

    for g in range(_RW // 32):
        r0 = base + g * 32 + lanei
        r1 = base + g * 32 + 16 + lanei
        pxi0 = plsc.load_gather(pxv, [r0])
        pyi0 = plsc.load_gather(pyv, [r0])
        vxi0 = plsc.load_gather(vxv, [r0])
        vyi0 = plsc.load_gather(vyv, [r0])
        pxi1 = plsc.load_gather(pxv, [r1])
        pyi1 = plsc.load_gather(pyv, [r1])
        vxi1 = plsc.load_gather(vxv, [r1])
        vyi1 = plsc.load_gather(vyv, [r1])

        zero_i = jnp.zeros((16,), jnp.int32)
        init = (tuple(jnp.full((16,), inf) for _ in range(_K)),
                tuple(zero_i for _ in range(_K)),
                tuple(jnp.full((16,), inf) for _ in range(_K)),
                tuple(zero_i for _ in range(_K)))

        def cascade(V, I, d2, js):
            nV, nI, mprev = [], [], None
            for t in range(_K):
                mle = V[t] <= d2
                iv = d2 if t == 0 else jnp.where(mprev, d2, V[t - 1])
                ii = js if t == 0 else jnp.where(mprev, js, I[t - 1])
                nV.append(jnp.where(mle, V[t], iv))
                nI.append(jnp.where(mle, I[t], ii))
                mprev = mle
            return nV, nI

        def chunk(jc, carry):
            Va, Ia, Vb, Ib = carry
            Va, Ia, Vb, Ib = list(Va), list(Ia), list(Vb), list(Ib)
            j0 = jc * 16
            for s in range(16):
                js = jnp.full((16,), j0 + s, jnp.int32)
                pxj = plsc.load_gather(pxv, [js])
                pyj = plsc.load_gather(pyv, [js])
                dxa = pxj - pxi0
                dya = pyj - pyi0
                d2a = jnp.where(r0 == js, inf, dxa * dxa + dya * dya)
                dxb = pxj - pxi1
                dyb = pyj - pyi1
                d2b = jnp.where(r1 == js, inf, dxb * dxb + dyb * dyb)
                Va, Ia = cascade(Va, Ia, d2a, js)
                Vb, Ib = cascade(Vb, Ib, d2b, js)
            return (tuple(Va), tuple(Ia), tuple(Vb), tuple(Ib))

        _, Ia, _, Ib = lax.fori_loop(0, _N // 16, chunk, init)

        for (I, pxi, pyi, vxi, vyi, roff) in (
                (Ia, pxi0, pyi0, vxi0, vyi0, g * 32),
                (Ib, pxi1, pyi1, vxi1, vyi1, g * 32 + 16)):
            rloc = roff + lanei
            for k in range(_K):
                gx = plsc.load_gather(pxv, [I[k]]) - pxi
                gy = plsc.load_gather(pyv, [I[k]]) - pyi
                gvx = plsc.load_gather(vxv, [I[k]]) - vxi
                gvy = plsc.load_gather(vyv, [I[k]]) - vyi
                for c, val in enumerate((gx, gy, gvx, gvy)):
                    col = jnp.full((16,), 4 * k + c, jnp.int32)
                    plsc.store_scatter(og, [rloc, col], val)

    pltpu.sync_copy(og, out_h.at[pl.ds(wid * _RW, _RW)])


def _sc_select(px, py, vx, vy):
    mesh = plsc.VectorSubcoreMesh(core_axis_name="c", subcore_axis_name="s")
    kfn = functools.partial(
        pl.kernel,
        mesh=mesh,
        compiler_params=pltpu.CompilerParams(needs_layout_passes=False),
        out_type=jax.ShapeDtypeStruct((_N - _S, 4 * _K), jnp.float32),
        scratch_types=[
            pltpu.VMEM((_N,), jnp.float32),
            pltpu.VMEM((_N,), jnp.float32),
            pltpu.VMEM((_N,), jnp.float32),
            pltpu.VMEM((_N,), jnp.float32),
            pltpu.VMEM((_RW, 4 * _K), jnp.float32),
        ],
    )(_sc_body)
    return kfn(px, py, vx, vy)


def _nt(a, b):
    return jax.lax.dot_general(a, b, (((1,), (1,)), ((), ())),
                               preferred_element_type=jnp.float32)


def _lstm_tail(x, Wih, bfull, oW, ob, out_ref):
    ig = jax.nn.sigmoid(_nt(x, Wih[0:_H, :]) + bfull[:, 0:_H])
    gg = jnp.tanh(_nt(x, Wih[2 * _H:3 * _H, :]) + bfull[:, 2 * _H:3 * _H])
    og = jax.nn.sigmoid(_nt(x, Wih[3 * _H:4 * _H, :]) + bfull[:, 3 * _H:4 * _H])
    h1 = og * jnp.tanh(ig * gg)
    out_ref[...] = _nt(h1, oW[...]) + ob[...]


def _tc_sel_body(o1T, o2T, o1b, o2b, pieces, embWT, embb, Wih, bfull, oW, ob,
                 out_ref):
    i = pl.program_id(0)
    pxj = o2T[0:1, :]
    pyj = o2T[1:2, :]
    vxj = pxj - o1T[0:1, :]
    vyj = pyj - o1T[1:2, :]
    pxi = o2b[:, 0:1]
    pyi = o2b[:, 1:2]
    vxi = pxi - o1b[:, 0:1]
    vyi = pyi - o1b[:, 1:2]
    dx = pxj - pxi
    dy = pyj - pyi
    d2 = dx * dx + dy * dy
    colio = jax.lax.broadcasted_iota(jnp.int32, (_R, _N), 1)
    rowg = i * _R + jax.lax.broadcasted_iota(jnp.int32, (_R, _N), 0)
    d2 = jnp.where(colio == rowg, jnp.inf, d2)
    xs = []
    for k in range(_K):
        m = jnp.min(d2, axis=1, keepdims=True)
        idx = jnp.min(jnp.where(d2 == m, colio, _N), axis=1, keepdims=True)
        sel = colio == idx
        onehot = jnp.where(sel, 1.0, 0.0).astype(jnp.bfloat16)
        if k + 1 < _K:
            d2 = jnp.where(sel, jnp.inf, d2)
        p = jnp.dot(onehot, pieces[...], preferred_element_type=jnp.float32)
        g = p[:, 0:4] + p[:, 4:8] + p[:, 8:12]
        e = ((g[:, 0:1] - pxi) * embWT[0:1, :] + (g[:, 1:2] - pyi) * embWT[1:2, :]
             + (g[:, 2:3] - vxi) * embWT[2:3, :] + (g[:, 3:4] - vyi) * embWT[3:4, :]
             + embb[...])
        xs.append(jnp.maximum(e, 0.0))
    x = jnp.concatenate(xs, axis=1)
    _lstm_tail(x, Wih, bfull, oW, ob, out_ref)


def _tc_grid_body(gridb, EW, eb, Wih, bfull, oW, ob, out_ref):
    x = jnp.maximum(
        jnp.dot(gridb[...], EW[...], preferred_element_type=jnp.float32)
        + eb[...], 0.0)
    _lstm_tail(x, Wih, bfull, oW, ob, out_ref)


def kernel(_hidden, obs1, obs2, emb_W, emb_b, W_ih, W_hh, b_ih, b_hh, out_W, out_b):
    del _hidden, W_hh
    vel = obs2 - obs1
    px, py = obs2[:, 0], obs2[:, 1]
    vx, vy = vel[:, 0], vel[:, 1]

    grid = _sc_select(px, py, vx, vy)

    embWT = emb_W.T
    embb = emb_b.reshape(1, -1)
    bfull = (b_ih + b_hh).reshape(1, -1)
    ob = out_b.reshape(1, -1)
    o1T = obs1.T
    o2T = obs2.T
    coords = jnp.concatenate([obs2, vel], axis=1)
    hi = coords.astype(jnp.bfloat16)
    r1 = coords - hi.astype(jnp.float32)
    mid = r1.astype(jnp.bfloat16)
    lo = (r1 - mid.astype(jnp.float32)).astype(jnp.bfloat16)
    pieces = jnp.concatenate([hi, mid, lo], axis=1)

    full = lambda shape: pl.BlockSpec(shape, lambda i: (0, 0))
    out1 = pl.pallas_call(
        _tc_sel_body,
        grid=(_S // _R,),
        in_specs=[
            full((2, _N)),
            full((2, _N)),
            pl.BlockSpec((_R, 2), lambda i: (i, 0)),
            pl.BlockSpec((_R, 2), lambda i: (i, 0)),
            full((_N, 12)),
            full((4, _OUT // _K)),
            full((1, _OUT // _K)),
            full((4 * _H, _OUT)),
            full((1, 4 * _H)),
            full((_OUT, _H)),
            full((1, _OUT)),
        ],
        out_specs=pl.BlockSpec((_R, _OUT), lambda i: (i, 0)),
        out_shape=jax.ShapeDtypeStruct((_S, _OUT), jnp.float32),
    )(o1T, o2T, obs1[:_S], obs2[:_S], pieces, embWT, embb, W_ih, bfull,
      out_W, ob)

    EW = jnp.kron(jnp.eye(_K, dtype=jnp.float32), emb_W.T)
    eb = jnp.tile(emb_b, _K).reshape(1, -1)
    out2 = pl.pallas_call(
        _tc_grid_body,
        grid=(1,),
        in_specs=[
            pl.BlockSpec((_N - _S, 4 * _K), lambda i: (i, 0)),
            full((4 * _K, _OUT)),
            full((1, _OUT)),
            full((4 * _H, _OUT)),
            full((1, 4 * _H)),
            full((_OUT, _H)),
            full((1, _OUT)),
        ],
        out_specs=pl.BlockSpec((_N - _S, _OUT), lambda i: (i, 0)),
        out_shape=jax.ShapeDtypeStruct((_N - _S, _OUT), jnp.float32),
    )(grid, EW, eb, W_ih, bfull, out_W, ob)

    return jnp.concatenate([out1, out2], axis=0)


# speedup vs baseline: 1.0029x; 1.0029x over previous
import functools

import jax
import jax.numpy as jnp
from jax import lax
from jax.experimental import pallas as pl
from jax.experimental.pallas import tpu as pltpu
from jax.experimental.pallas import tpu_sc as plsc

_N = 2048
_S = 1024
_R = 512
_H = 1024
_OUT = 256
_K = 8
_RW = (_N - _S) // 32


def _sc_body(px_h, py_h, vx_h, vy_h, out_h, pxv, pyv, vxv, vyv, og):
    wid = lax.axis_index("s") * 2 + lax.axis_index("c")
    pltpu.sync_copy(px_h, pxv)
    pltpu.sync_copy(py_h, pyv)
    pltpu.sync_copy(vx_h, vxv)
    pltpu.sync_copy(vy_h, vyv)
    base = _S + wid * _RW
    lanei = lax.iota(jnp.int32, 16)
    inf = jnp.float32(jnp.inf)

    for g in range(_RW // 32):
        r0 = base + g * 32 + lanei
        r1 = base + g * 32 + 16 + lanei
        pxi0 = plsc.load_gather(pxv, [r0])
        pyi0 = plsc.load_gather(pyv, [r0])
        vxi0 = plsc.load_gather(vxv, [r0])
        vyi0 = plsc.load_gather(vyv, [r0])
        pxi1 = plsc.load_gather(pxv, [r1])
        pyi1 = plsc.load_gather(pyv, [r1])
        vxi1 = plsc.load_gather(vxv, [r1])
        vyi1 = plsc.load_gather(vyv, [r1])

        zero_i = jnp.zeros((16,), jnp.int32)
        init = (tuple(jnp.full((16,), inf) for _ in range(_K)),
                tuple(zero_i for _ in range(_K)),
                tuple(jnp.full((16,), inf) for _ in range(_K)),
                tuple(zero_i for _ in range(_K)))

        def cascade(V, I, d2, js):
            nV, nI, mprev = [], [], None
            for t in range(_K):
                mle = V[t] <= d2
                iv = d2 if t == 0 else jnp.where(mprev, d2, V[t - 1])
                ii = js if t == 0 else jnp.where(mprev, js, I[t - 1])
                nV.append(jnp.where(mle, V[t], iv))
                nI.append(jnp.where(mle, I[t], ii))
                mprev = mle
            return nV, nI

        def chunk(jc, carry):
            Va, Ia, Vb, Ib = carry
            Va, Ia, Vb, Ib = list(Va), list(Ia), list(Vb), list(Ib)
            j0 = jc * 16
            for s in range(16):
                js = jnp.full((16,), j0 + s, jnp.int32)
                pxj = plsc.load_gather(pxv, [js])
                pyj = plsc.load_gather(pyv, [js])
                dxa = pxj - pxi0
                dya = pyj - pyi0
                d2a = jnp.where(r0 == js, inf, dxa * dxa + dya * dya)
                dxb = pxj - pxi1
                dyb = pyj - pyi1
                d2b = jnp.where(r1 == js, inf, dxb * dxb + dyb * dyb)
                Va, Ia = cascade(Va, Ia, d2a, js)
                Vb, Ib = cascade(Vb, Ib, d2b, js)
            return (tuple(Va), tuple(Ia), tuple(Vb), tuple(Ib))

        _, Ia, _, Ib = lax.fori_loop(0, _N // 16, chunk, init)

        for (I, pxi, pyi, vxi, vyi, roff) in (
                (Ia, pxi0, pyi0, vxi0, vyi0, g * 32),
                (Ib, pxi1, pyi1, vxi1, vyi1, g * 32 + 16)):
            rloc = roff + lanei
            for k in range(_K):
                gx = plsc.load_gather(pxv, [I[k]]) - pxi
                gy = plsc.load_gather(pyv, [I[k]]) - pyi
                gvx = plsc.load_gather(vxv, [I[k]]) - vxi
                gvy = plsc.load_gather(vyv, [I[k]]) - vyi
                for c, val in enumerate((gx, gy, gvx, gvy)):
                    col = jnp.full((16,), 4 * k + c, jnp.int32)
                    plsc.store_scatter(og, [rloc, col], val)

    pltpu.sync_copy(og, out_h.at[pl.ds(wid * _RW, _RW)])


def _sc_select(px, py, vx, vy):
    mesh = plsc.VectorSubcoreMesh(core_axis_name="c", subcore_axis_name="s")
    kfn = functools.partial(
        pl.kernel,
        mesh=mesh,
        compiler_params=pltpu.CompilerParams(needs_layout_passes=False),
        out_type=jax.ShapeDtypeStruct((_N - _S, 4 * _K), jnp.float32),
        scratch_types=[
            pltpu.VMEM((_N,), jnp.float32),
            pltpu.VMEM((_N,), jnp.float32),
            pltpu.VMEM((_N,), jnp.float32),
            pltpu.VMEM((_N,), jnp.float32),
            pltpu.VMEM((_RW, 4 * _K), jnp.float32),
        ],
    )(_sc_body)
    return kfn(px, py, vx, vy)


def _nt(a, b):
    return jax.lax.dot_general(a, b, (((1,), (1,)), ((), ())),
                               preferred_element_type=jnp.float32)


def _lstm_tail(x, Wih, bfull, oW, ob, out_ref):
    ig = jax.nn.sigmoid(_nt(x, Wih[0:_H, :]) + bfull[:, 0:_H])
    gg = jnp.tanh(_nt(x, Wih[2 * _H:3 * _H, :]) + bfull[:, 2 * _H:3 * _H])
    og = jax.nn.sigmoid(_nt(x, Wih[3 * _H:4 * _H, :]) + bfull[:, 3 * _H:4 * _H])
    h1 = og * jnp.tanh(ig * gg)
    out_ref[...] = _nt(h1, oW[...]) + ob[...]


def _tc_sel_body(o1T, o2T, o1b, o2b, pieces, embWT, embb, Wih, bfull, oW, ob,
                 out_ref):
    i = pl.program_id(0)
    pxj = o2T[0:1, :]
    pyj = o2T[1:2, :]
    vxj = pxj - o1T[0:1, :]
    vyj = pyj - o1T[1:2, :]
    pxi = o2b[:, 0:1]
    pyi = o2b[:, 1:2]
    vxi = pxi - o1b[:, 0:1]
    vyi = pyi - o1b[:, 1:2]
    dx = pxj - pxi
    dy = pyj - pyi
    d2 = dx * dx + dy * dy
    colio = jax.lax.broadcasted_iota(jnp.int32, (_R, _N), 1)
    rowg = i * _R + jax.lax.broadcasted_iota(jnp.int32, (_R, _N), 0)
    d2 = jnp.where(colio == rowg, jnp.inf, d2)
    xs = []
    for k in range(_K):
        m = jnp.min(d2, axis=1, keepdims=True)
        idx = jnp.min(jnp.where(d2 == m, colio, _N), axis=1, keepdims=True)
        sel = colio == idx
        onehot = jnp.where(sel, 1.0, 0.0).astype(jnp.bfloat16)
        if k + 1 < _K:
            d2 = jnp.where(sel, jnp.inf, d2)
        p = jnp.dot(onehot, pieces[...], preferred_element_type=jnp.float32)
        g = p[:, 0:4] + p[:, 4:8] + p[:, 8:12]
        e = ((g[:, 0:1] - pxi) * embWT[0:1, :] + (g[:, 1:2] - pyi) * embWT[1:2, :]
             + (g[:, 2:3] - vxi) * embWT[2:3, :] + (g[:, 3:4] - vyi) * embWT[3:4, :]
             + embb[...])
        xs.append(jnp.maximum(e, 0.0))
    x = jnp.concatenate(xs, axis=1)
    _lstm_tail(x, Wih, bfull, oW, ob, out_ref)


def _tc_grid_body(gridb, EW, eb, Wih, bfull, oW, ob, out_ref):
    x = jnp.maximum(
        jnp.dot(gridb[...], EW[...], preferred_element_type=jnp.float32)
        + eb[...], 0.0)
    _lstm_tail(x, Wih, bfull, oW, ob, out_ref)


def kernel(_hidden, obs1, obs2, emb_W, emb_b, W_ih, W_hh, b_ih, b_hh, out_W, out_b):
    del _hidden, W_hh
    vel = obs2 - obs1
    px, py = obs2[:, 0], obs2[:, 1]
    vx, vy = vel[:, 0], vel[:, 1]

    grid = _sc_select(px, py, vx, vy)

    embWT = emb_W.T
    embb = emb_b.reshape(1, -1)
    bfull = (b_ih + b_hh).reshape(1, -1)
    ob = out_b.reshape(1, -1)
    o1T = obs1.T
    o2T = obs2.T
    coords = jnp.concatenate([obs2, vel], axis=1)
    hi = coords.astype(jnp.bfloat16)
    r1 = coords - hi.astype(jnp.float32)
    mid = r1.astype(jnp.bfloat16)
    lo = (r1 - mid.astype(jnp.float32)).astype(jnp.bfloat16)
    pieces = jnp.concatenate([hi, mid, lo], axis=1)

    full = lambda shape: pl.BlockSpec(shape, lambda i: (0, 0))
    out1 = pl.pallas_call(
        _tc_sel_body,
        grid=(_S // _R,),
        in_specs=[
            full((2, _N)),
            full((2, _N)),
            pl.BlockSpec((_R, 2), lambda i: (i, 0)),
            pl.BlockSpec((_R, 2), lambda i: (i, 0)),
            full((_N, 12)),
            full((4, _OUT // _K)),
            full((1, _OUT // _K)),
            full((4 * _H, _OUT)),
            full((1, 4 * _H)),
            full((_OUT, _H)),
            full((1, _OUT)),
        ],
        out_specs=pl.BlockSpec((_R, _OUT), lambda i: (i, 0)),
        out_shape=jax.ShapeDtypeStruct((_S, _OUT), jnp.float32),
    )(o1T, o2T, obs1[:_S], obs2[:_S], pieces, embWT, embb, W_ih, bfull,
      out_W, ob)

    EW = jnp.kron(jnp.eye(_K, dtype=jnp.float32), emb_W.T)
    eb = jnp.tile(emb_b, _K).reshape(1, -1)
    out2 = pl.pallas_call(
        _tc_grid_body,
        grid=((_N - _S) // _R,),
        in_specs=[
            pl.BlockSpec((_R, 4 * _K), lambda i: (i, 0)),
            full((4 * _K, _OUT)),
            full((1, _OUT)),
            full((4 * _H, _OUT)),
            full((1, 4 * _H)),
            full((_OUT, _H)),
            full((1, _OUT)),
        ],
        out_specs=pl.BlockSpec((_R, _OUT), lambda i: (i, 0)),
        out_shape=jax.ShapeDtypeStruct((_N - _S, _OUT), jnp.float32),
    )(grid, EW, eb, W_ih, bfull, out_W, ob)

    return jnp.concatenate([out1, out2], axis=0)
